# Initial kernel scaffold; baseline (speedup 1.0000x reference)
import jax
import jax.numpy as jnp
from jax.experimental import pallas as pl


def kernel(features, labels, anchor_feature, anchor_label):
    raise NotImplementedError("write your pallas kernel here")


# speedup vs baseline: 10.1335x; 10.1335x over previous
import functools

import jax
import jax.numpy as jnp
from jax import lax
from jax.experimental import pallas as pl
from jax.experimental.pallas import tpu as pltpu
from jax.experimental.pallas import tpu_sc as plsc

B = 4096
D = 128
K = 100000
GR = 50
S = 64
PADK = 100352
C = PADK // S
RB = 256
AB = 2048
J = 56
JP = 64
TOT = B * JP
NEG = -1.0e30
LBL_SHIFT = 131072

NC, NS = 2, 16
NW = NC * NS
PER_W = TOT // NW
CH = 128
NCH = PER_W // CH


def _k1_body(f_ref, a_ref, sims_ref, cmax_ref):
    j = pl.program_id(1)
    x = lax.dot_general(f_ref[...], a_ref[...], (((1,), (1,)), ((), ())),
                        preferred_element_type=jnp.float32)
    col = j * AB + lax.broadcasted_iota(jnp.int32, (RB, AB), 1)
    x = jnp.where(col < K, x, NEG)
    sims_ref[...] = x
    cmax_ref[...] = jnp.max(x.reshape(RB, AB // S, S), axis=2)[None]


_k1 = pl.pallas_call(
    _k1_body,
    grid=(B // RB, PADK // AB),
    in_specs=[
        pl.BlockSpec((RB, D), lambda i, j: (i, 0)),
        pl.BlockSpec((AB, D), lambda i, j: (j, 0)),
    ],
    out_specs=[
        pl.BlockSpec((RB, AB), lambda i, j: (i, j)),
        pl.BlockSpec((1, RB, AB // S), lambda i, j: (j, i, 0)),
    ],
    out_shape=[
        jax.ShapeDtypeStruct((B, PADK), jnp.float32),
        jax.ShapeDtypeStruct((PADK // AB, B, AB // S), jnp.float32),
    ],
)


def _k2_body(cmax_ref, gidx_ref, cidx_ref):
    i = pl.program_id(0)
    x3 = cmax_ref[...]
    x = jnp.concatenate([x3[j] for j in range(PADK // AB)], axis=1)
    iota = lax.broadcasted_iota(jnp.int32, (RB, C), 1)
    cols = []
    for _ in range(J):
        m = jnp.max(x, axis=1, keepdims=True)
        cand = jnp.where(x == m, iota, C)
        a = jnp.min(cand, axis=1, keepdims=True)
        x = jnp.where(iota == a, NEG, x)
        cols.append(a)
    cols.append(jnp.full((RB, JP - J), C - 1, jnp.int32))
    cidx = jnp.concatenate(cols, axis=1)
    rows = i * RB + lax.broadcasted_iota(jnp.int32, (RB, JP), 0)
    cidx_ref[...] = cidx
    gidx_ref[...] = rows * C + cidx


_k2 = pl.pallas_call(
    _k2_body,
    grid=(B // RB,),
    in_specs=[pl.BlockSpec((PADK // AB, RB, AB // S), lambda i: (0, i, 0))],
    out_specs=[
        pl.BlockSpec((RB, JP), lambda i: (i, 0)),
        pl.BlockSpec((RB, JP), lambda i: (i, 0)),
    ],
    out_shape=[
        jax.ShapeDtypeStruct((B, JP), jnp.int32),
        jax.ShapeDtypeStruct((B, JP), jnp.int32),
    ],
)


def _k3_body(gidx_hbm, cidx_hbm, simsrows_hbm, coderows_hbm,
             osims_hbm, ocode_hbm, gv, cv, sbuf, cbuf, sem1, sem2):
    wid = lax.axis_index("s") * NC + lax.axis_index("c")

    def step(k, carry):
        base = wid * PER_W + k * CH
        pltpu.sync_copy(gidx_hbm.at[pl.ds(base, CH)], gv)
        pltpu.sync_copy(cidx_hbm.at[pl.ds(base, CH)], cv)
        cp1 = pltpu.async_copy(simsrows_hbm.at[gv], sbuf, sem1)
        cp2 = pltpu.async_copy(coderows_hbm.at[cv], cbuf, sem2)
        cp1.wait()
        cp2.wait()
        pltpu.sync_copy(sbuf, osims_hbm.at[pl.ds(base, CH)])
        pltpu.sync_copy(cbuf, ocode_hbm.at[pl.ds(base, CH)])
        return carry

    lax.fori_loop(0, NCH, step, 0)


@functools.cache
def _k3():
    return pl.kernel(
        _k3_body,
        out_type=(
            jax.ShapeDtypeStruct((TOT, S), jnp.float32),
            jax.ShapeDtypeStruct((TOT, S), jnp.int32),
        ),
        mesh=plsc.VectorSubcoreMesh(core_axis_name="c", subcore_axis_name="s",
                                    num_cores=NC, num_subcores=NS),
        compiler_params=pltpu.CompilerParams(use_tc_tiling_on_sc=False),
        scratch_types=[
            pltpu.VMEM((CH,), jnp.int32),
            pltpu.VMEM((CH,), jnp.int32),
            pltpu.VMEM((CH, S), jnp.float32),
            pltpu.VMEM((CH, S), jnp.int32),
            pltpu.SemaphoreType.DMA,
            pltpu.SemaphoreType.DMA,
        ],
    )


def _k4_body(s_ref, c_ref, lab_ref, out_ref):
    i = pl.program_id(0)
    x = s_ref[...]
    code = c_ref[...]
    qlbl = lab_ref[...]
    real = x > -1.0e29
    hi0 = jnp.max(x, axis=1, keepdims=True) + 1.0
    lo0 = jnp.min(jnp.where(real, x, 1.0e30), axis=1, keepdims=True) - 1.0

    def vbis(_, carry):
        lo, hi = carry
        mid = 0.5 * (lo + hi)
        cnt = jnp.sum((x > mid).astype(jnp.int32), axis=1, keepdims=True)
        p = cnt >= GR
        return jnp.where(p, mid, lo), jnp.where(p, hi, mid)

    lo, hi = lax.fori_loop(0, 34, vbis, (lo0, hi0))

    gt = x > hi
    tie = (x > lo) & jnp.logical_not(gt)
    aid = code & (LBL_SHIFT - 1)
    albl = lax.shift_right_logical(code, 17)
    match = albl == qlbl

    cnt_gt = jnp.sum(gt.astype(jnp.int32), axis=1, keepdims=True)
    slots = GR - cnt_gt

    def ibis(_, carry):
        lo_i, hi_i = carry
        mid = (lo_i + hi_i) // 2
        cnt = jnp.sum((tie & (aid <= mid)).astype(jnp.int32),
                      axis=1, keepdims=True)
        p = cnt >= slots
        return jnp.where(p, lo_i, mid), jnp.where(p, mid, hi_i)

    lo_i0 = jnp.full_like(cnt_gt, -1)
    hi_i0 = jnp.full_like(cnt_gt, PADK - 1)
    _, cut = lax.fori_loop(0, 18, ibis, (lo_i0, hi_i0))

    m_gt = jnp.sum((gt & match).astype(jnp.int32), axis=1, keepdims=True)
    m_tie = jnp.sum((tie & match & (aid <= cut)).astype(jnp.int32),
                    axis=1, keepdims=True)
    matches = m_gt + m_tie
    blocksum = jnp.sum(matches.astype(jnp.float32), axis=0, keepdims=True)

    @pl.when(i == 0)
    def _():
        out_ref[...] = jnp.zeros((1, 1), jnp.float32)

    out_ref[...] += blocksum * (-1.0 / (GR * B))


_k4 = pl.pallas_call(
    _k4_body,
    grid=(B // RB,),
    in_specs=[
        pl.BlockSpec((RB, JP * S), lambda i: (i, 0)),
        pl.BlockSpec((RB, JP * S), lambda i: (i, 0)),
        pl.BlockSpec((RB, 1), lambda i: (i, 0)),
    ],
    out_specs=pl.BlockSpec((1, 1), lambda i: (0, 0)),
    out_shape=jax.ShapeDtypeStruct((1, 1), jnp.float32),
)


def kernel(features, labels, anchor_feature, anchor_label):
    labels = labels.astype(jnp.int32)
    anchor_label = anchor_label.astype(jnp.int32)
    a_pad = jnp.pad(anchor_feature, ((0, PADK - K), (0, 0)))
    sims, cmax = _k1(features, a_pad)
    gidx, cidx = _k2(cmax)
    code_tab = (jnp.pad(anchor_label, (0, PADK - K)) * LBL_SHIFT
                + jnp.arange(PADK, dtype=jnp.int32)).reshape(C, S)
    gsims, gcode = _k3()(gidx.reshape(-1), cidx.reshape(-1),
                         sims.reshape(B * C, S), code_tab)
    out = _k4(gsims.reshape(B, JP * S), gcode.reshape(B, JP * S),
              labels.reshape(B, 1))
    return out.reshape(())


# speedup vs baseline: 11.0552x; 1.0910x over previous
import functools

import jax
import jax.numpy as jnp
from jax import lax
from jax.experimental import pallas as pl
from jax.experimental.pallas import tpu as pltpu
from jax.experimental.pallas import tpu_sc as plsc

B = 4096
D = 128
K = 100000
GR = 50
S = 128
PADK = 100352
C = PADK // S
RB = 256
AB = 2048
NJ = PADK // AB
J = 56
W = J * S
TOT = B * J
NEG = -1.0e30
LBL_SHIFT = 131072

NC, NS = 2, 16
NW = NC * NS
PER_W = TOT // NW
CH = 128
NCH = PER_W // CH


def _k1_body(f_ref, a_ref, sims_ref, cmax_ref):
    j = pl.program_id(1)
    x = lax.dot_general(f_ref[...], a_ref[...], (((1,), (1,)), ((), ())),
                        preferred_element_type=jnp.float32)
    col = j * AB + lax.broadcasted_iota(jnp.int32, (RB, AB), 1)
    x = jnp.where(col < K, x, NEG)
    sims_ref[...] = x
    cmax_ref[...] = jnp.max(x.reshape(RB, AB // S, S), axis=2)[None]


_k1 = pl.pallas_call(
    _k1_body,
    grid=(B // RB, NJ),
    in_specs=[
        pl.BlockSpec((RB, D), lambda i, j: (i, 0)),
        pl.BlockSpec((AB, D), lambda i, j: (j, 0)),
    ],
    out_specs=[
        pl.BlockSpec((RB, AB), lambda i, j: (i, j)),
        pl.BlockSpec((1, RB, AB // S), lambda i, j: (j, i, 0)),
    ],
    out_shape=[
        jax.ShapeDtypeStruct((B, PADK), jnp.float32),
        jax.ShapeDtypeStruct((NJ, B, AB // S), jnp.float32),
    ],
)


def _k2_body(cmax_ref, gidx_ref, cidx_ref):
    i = pl.program_id(0)
    x3 = cmax_ref[...]
    x = jnp.concatenate([x3[j] for j in range(NJ)], axis=1)
    iota = lax.broadcasted_iota(jnp.int32, (RB, C), 1)
    cols = []
    for _ in range(J):
        m = jnp.max(x, axis=1, keepdims=True)
        cand = jnp.where(x == m, iota, C)
        a = jnp.min(cand, axis=1, keepdims=True)
        x = jnp.where(iota == a, NEG, x)
        cols.append(a)
    cidx = jnp.concatenate(cols, axis=1)
    rows = i * RB + lax.broadcasted_iota(jnp.int32, (RB, J), 0)
    cidx_ref[...] = cidx
    gidx_ref[...] = rows * C + cidx


_k2 = pl.pallas_call(
    _k2_body,
    grid=(B // RB,),
    in_specs=[pl.BlockSpec((NJ, RB, AB // S), lambda i: (0, i, 0))],
    out_specs=[
        pl.BlockSpec((RB, J), lambda i: (i, 0)),
        pl.BlockSpec((RB, J), lambda i: (i, 0)),
    ],
    out_shape=[
        jax.ShapeDtypeStruct((B, J), jnp.int32),
        jax.ShapeDtypeStruct((B, J), jnp.int32),
    ],
)


def _k3_body(gidx_hbm, cidx_hbm, simsrows_hbm, coderows_hbm,
             osims_hbm, ocode_hbm, gv0, gv1, cv0, cv1,
             sbuf0, sbuf1, cbuf0, cbuf1,
             gsem0, gsem1, csem0, csem1):
    wid = lax.axis_index("s") * NC + lax.axis_index("c")
    wbase = wid * PER_W
    gvs, cvs = (gv0, gv1), (cv0, cv1)
    sbufs, cbufs = (sbuf0, sbuf1), (cbuf0, cbuf1)
    gsems, csems = (gsem0, gsem1), (csem0, csem1)

    def step(t, carry):
        base0 = wbase + t * (2 * CH)
        cps = []
        for b in range(2):
            base = base0 + b * CH
            pltpu.sync_copy(gidx_hbm.at[pl.ds(base, CH)], gvs[b])
            pltpu.sync_copy(cidx_hbm.at[pl.ds(base, CH)], cvs[b])
            cps.append((
                pltpu.async_copy(simsrows_hbm.at[gvs[b]], sbufs[b], gsems[b]),
                pltpu.async_copy(coderows_hbm.at[cvs[b]], cbufs[b], csems[b]),
                base,
            ))
        for b in range(2):
            cp1, cp2, base = cps[b]
            cp1.wait()
            cp2.wait()
            pltpu.sync_copy(sbufs[b], osims_hbm.at[pl.ds(base, CH)])
            pltpu.sync_copy(cbufs[b], ocode_hbm.at[pl.ds(base, CH)])
        return carry

    lax.fori_loop(0, NCH // 2, step, 0)


@functools.cache
def _k3():
    return pl.kernel(
        _k3_body,
        out_type=(
            jax.ShapeDtypeStruct((TOT, S), jnp.float32),
            jax.ShapeDtypeStruct((TOT, S), jnp.int32),
        ),
        mesh=plsc.VectorSubcoreMesh(core_axis_name="c", subcore_axis_name="s",
                                    num_cores=NC, num_subcores=NS),
        scratch_types=[
            pltpu.VMEM((CH,), jnp.int32),
            pltpu.VMEM((CH,), jnp.int32),
            pltpu.VMEM((CH,), jnp.int32),
            pltpu.VMEM((CH,), jnp.int32),
            pltpu.VMEM((CH, S), jnp.float32),
            pltpu.VMEM((CH, S), jnp.float32),
            pltpu.VMEM((CH, S), jnp.int32),
            pltpu.VMEM((CH, S), jnp.int32),
            pltpu.SemaphoreType.DMA,
            pltpu.SemaphoreType.DMA,
            pltpu.SemaphoreType.DMA,
            pltpu.SemaphoreType.DMA,
        ],
    )


def _k4_body(s_ref, c_ref, lab_ref, out_ref):
    i = pl.program_id(0)
    x = s_ref[...]
    code = c_ref[...]
    qlbl = lab_ref[...]
    real = x > -1.0e29
    hi0 = jnp.max(x, axis=1, keepdims=True) + 1.0
    lo0 = jnp.min(jnp.where(real, x, 1.0e30), axis=1, keepdims=True) - 1.0

    def vbis(_, carry):
        lo, hi = carry
        mid = 0.5 * (lo + hi)
        cnt = jnp.sum((x > mid).astype(jnp.int32), axis=1, keepdims=True)
        p = cnt >= GR
        return jnp.where(p, mid, lo), jnp.where(p, hi, mid)

    lo, hi = lax.fori_loop(0, 34, vbis, (lo0, hi0))

    gt = x > hi
    tie = (x > lo) & jnp.logical_not(gt)
    aid = code & (LBL_SHIFT - 1)
    albl = lax.shift_right_logical(code, 17)
    match = albl == qlbl

    cnt_gt = jnp.sum(gt.astype(jnp.int32), axis=1, keepdims=True)
    slots = GR - cnt_gt

    def ibis(_, carry):
        lo_i, hi_i = carry
        mid = (lo_i + hi_i) // 2
        cnt = jnp.sum((tie & (aid <= mid)).astype(jnp.int32),
                      axis=1, keepdims=True)
        p = cnt >= slots
        return jnp.where(p, lo_i, mid), jnp.where(p, mid, hi_i)

    lo_i0 = jnp.full_like(cnt_gt, -1)
    hi_i0 = jnp.full_like(cnt_gt, PADK - 1)
    _, cut = lax.fori_loop(0, 18, ibis, (lo_i0, hi_i0))

    m_gt = jnp.sum((gt & match).astype(jnp.int32), axis=1, keepdims=True)
    m_tie = jnp.sum((tie & match & (aid <= cut)).astype(jnp.int32),
                    axis=1, keepdims=True)
    matches = m_gt + m_tie
    blocksum = jnp.sum(matches.astype(jnp.float32), axis=0, keepdims=True)

    @pl.when(i == 0)
    def _():
        out_ref[...] = jnp.zeros((1, 1), jnp.float32)

    out_ref[...] += blocksum * (-1.0 / (GR * B))


_k4 = pl.pallas_call(
    _k4_body,
    grid=(B // RB,),
    in_specs=[
        pl.BlockSpec((RB, W), lambda i: (i, 0)),
        pl.BlockSpec((RB, W), lambda i: (i, 0)),
        pl.BlockSpec((RB, 1), lambda i: (i, 0)),
    ],
    out_specs=pl.BlockSpec((1, 1), lambda i: (0, 0)),
    out_shape=jax.ShapeDtypeStruct((1, 1), jnp.float32),
)


def kernel(features, labels, anchor_feature, anchor_label):
    labels = labels.astype(jnp.int32)
    anchor_label = anchor_label.astype(jnp.int32)
    a_pad = jnp.pad(anchor_feature, ((0, PADK - K), (0, 0)))
    sims, cmax = _k1(features, a_pad)
    gidx, cidx = _k2(cmax)
    code_tab = (jnp.pad(anchor_label, (0, PADK - K)) * LBL_SHIFT
                + jnp.arange(PADK, dtype=jnp.int32)).reshape(C, S)
    gsims, gcode = _k3()(gidx.reshape(-1), cidx.reshape(-1),
                         sims.reshape(B * C, S), code_tab)
    out = _k4(gsims.reshape(B, W), gcode.reshape(B, W),
              labels.reshape(B, 1))
    return out.reshape(())
